# baseline (speedup 1.0000x reference)
import functools

import jax
import jax.numpy as jnp
from jax import lax
from jax.experimental import pallas as pl
from jax.experimental.pallas import tpu as pltpu
from jax.experimental.pallas import tpu_sc as plsc

_B = 16384
_D = 64
_INFO = plsc.get_sparse_core_info()
_NC = _INFO.num_cores
_NS = _INFO.num_subcores
_L = _INFO.num_lanes
_NW = _NC * _NS
_BPW = _B // _NW
_HALF = _BPW // 2
_CH = 128
_G = 16


def _distmult_body(h_idx_hbm, t_idx_hbm, r_idx_hbm, node_hbm, rel_hbm,
                   out_hbm, hi_v, ti_v, ri_v, hp_v, tp_v, rp_v,
                   h_rows, t_rows, r_rows, scores_v, sem):
    wid = lax.axis_index("s") * _NC + lax.axis_index("c")
    base = wid * _BPW

    pltpu.sync_copy(h_idx_hbm.at[pl.ds(base, _BPW)], hi_v)
    pltpu.sync_copy(t_idx_hbm.at[pl.ds(base, _BPW)], ti_v)
    pltpu.sync_copy(r_idx_hbm.at[pl.ds(base, _BPW)], ri_v)

    for v in range(_BPW // _L):
        sl = pl.ds(v * _L, _L)
        hp_v[sl] = lax.shift_right_logical(hi_v[sl], 1)
        tp_v[sl] = lax.shift_right_logical(ti_v[sl], 1)
        rp_v[sl] = lax.shift_right_logical(ri_v[sl], 1)

    lane = lax.iota(jnp.int32, _L)

    for half in range(2):
        hoff = half * _HALF
        copies = []
        for c in range(_HALF // _CH):
            src = pl.ds(hoff + c * _CH, _CH)
            dst = pl.ds(c * _CH, _CH)
            copies.append(pltpu.async_copy(
                node_hbm.at[hp_v.at[src]], h_rows.at[dst], sem))
            copies.append(pltpu.async_copy(
                node_hbm.at[tp_v.at[src]], t_rows.at[dst], sem))
            copies.append(pltpu.async_copy(
                rel_hbm.at[rp_v.at[src]], r_rows.at[dst], sem))
        for cp in copies:
            cp.wait()

        def group(g, carry):
            rb = g * _G
            sl_g = pl.ds(hoff + rb, _G)
            hov = (hi_v[sl_g] & 1) * _D
            tov = (ti_v[sl_g] & 1) * _D
            rov = (ri_v[sl_g] & 1) * _D
            acc = jnp.zeros((_L,), jnp.float32)
            for j in range(_G):
                row = rb + j
                ho = hov[j]
                to = tov[j]
                ro = rov[j]
                p = None
                for k in range(_D // _L):
                    prod = (h_rows[row, pl.ds(ho + k * _L, _L)]
                            * r_rows[row, pl.ds(ro + k * _L, _L)]
                            * t_rows[row, pl.ds(to + k * _L, _L)])
                    p = prod if p is None else p + prod
                acc = jnp.where(lane == j, jnp.sum(p), acc)
            scores_v[pl.ds(rb, _G)] = acc
            return carry

        lax.fori_loop(0, _HALF // _G, group, 0)
        pltpu.sync_copy(scores_v, out_hbm.at[pl.ds(base + hoff, _HALF)])


@functools.partial(
    pl.kernel,
    out_type=jax.ShapeDtypeStruct((_B,), jnp.float32),
    mesh=plsc.VectorSubcoreMesh(core_axis_name="c", subcore_axis_name="s"),
    compiler_params=pltpu.CompilerParams(needs_layout_passes=False,
                                         use_tc_tiling_on_sc=True),
    scratch_types=[
        pltpu.VMEM((_BPW,), jnp.int32),
        pltpu.VMEM((_BPW,), jnp.int32),
        pltpu.VMEM((_BPW,), jnp.int32),
        pltpu.VMEM((_BPW,), jnp.int32),
        pltpu.VMEM((_BPW,), jnp.int32),
        pltpu.VMEM((_BPW,), jnp.int32),
        pltpu.VMEM((_HALF, 2 * _D), jnp.float32),
        pltpu.VMEM((_HALF, 2 * _D), jnp.float32),
        pltpu.VMEM((_HALF, 2 * _D), jnp.float32),
        pltpu.VMEM((_HALF,), jnp.float32),
        pltpu.SemaphoreType.DMA,
    ],
)
def _distmult_sc(h_idx, t_idx, r_idx, node_emb, rel_emb, out,
                 hi_v, ti_v, ri_v, hp_v, tp_v, rp_v,
                 h_rows, t_rows, r_rows, scores_v, sem):
    _distmult_body(h_idx, t_idx, r_idx, node_emb, rel_emb, out,
                   hi_v, ti_v, ri_v, hp_v, tp_v, rp_v,
                   h_rows, t_rows, r_rows, scores_v, sem)


def kernel(head_indices, tail_indices, relation_indices, node_embedding,
           relation_embedding):
    node2 = node_embedding.reshape(node_embedding.shape[0] // 2, 2 * _D)
    rel2 = relation_embedding.reshape(relation_embedding.shape[0] // 2,
                                      2 * _D)
    return _distmult_sc(head_indices.astype(jnp.int32),
                        tail_indices.astype(jnp.int32),
                        relation_indices.astype(jnp.int32),
                        node2, rel2)
